# baseline (speedup 1.0000x reference)
import functools

import jax
import jax.numpy as jnp
from jax.experimental import pallas as pl
from jax.experimental.pallas import tpu as pltpu

_LANES = 128


def _head_kernel(self_ref, nb_ref, inv_ref, ws_ref, wn_ref, b_ref, out_ref,
                 *, n_nb, hidden):
    parts = [nb_ref[:, n * hidden:(n + 1) * hidden] for n in range(n_nb)]
    while len(parts) > 1:
        nxt = [parts[i] + parts[i + 1] for i in range(0, len(parts) - 1, 2)]
        if len(parts) % 2:
            nxt.append(parts[-1])
        parts = nxt
    nb_sum = parts[0]

    acc = jnp.dot(self_ref[...], ws_ref[...],
                  preferred_element_type=jnp.float32)
    acc += jnp.dot(nb_sum, wn_ref[...],
                   preferred_element_type=jnp.float32) * inv_ref[...]
    out_ref[...] = acc + b_ref[...]


def kernel(emb_self, emb_nb, weight, bias, nb_counts):
    B, H = emb_self.shape
    _, N, _ = emb_nb.shape

    wt = weight.astype(jnp.float32)
    w_self = jnp.zeros((H, _LANES), jnp.float32).at[:, :2].set(wt[:, :H].T)
    w_nb = jnp.zeros((H, _LANES), jnp.float32).at[:, :2].set(wt[:, H:].T)
    b_pad = jnp.zeros((1, _LANES), jnp.float32).at[0, :2].set(
        bias.astype(jnp.float32))

    inv_cnt = (1.0 / jnp.maximum(nb_counts.astype(jnp.float32), 1.0)
               ).reshape(B, 1)

    nb_2d = emb_nb.reshape(B, N * H)

    bt = min(B, 256)
    grid = (pl.cdiv(B, bt),)

    out_pad = pl.pallas_call(
        functools.partial(_head_kernel, n_nb=N, hidden=H),
        out_shape=jax.ShapeDtypeStruct((B, _LANES), jnp.float32),
        grid_spec=pl.GridSpec(
            grid=grid,
            in_specs=[
                pl.BlockSpec((bt, H), lambda i: (i, 0)),
                pl.BlockSpec((bt, N * H), lambda i: (i, 0)),
                pl.BlockSpec((bt, 1), lambda i: (i, 0)),
                pl.BlockSpec((H, _LANES), lambda i: (0, 0)),
                pl.BlockSpec((H, _LANES), lambda i: (0, 0)),
                pl.BlockSpec((1, _LANES), lambda i: (0, 0)),
            ],
            out_specs=pl.BlockSpec((bt, _LANES), lambda i: (i, 0)),
        ),
        compiler_params=pltpu.CompilerParams(
            dimension_semantics=("parallel",),
            vmem_limit_bytes=64 << 20),
        cost_estimate=pl.CostEstimate(
            flops=2 * B * H * _LANES * 2 + B * (N - 1) * H,
            transcendentals=0,
            bytes_accessed=B * (N + 1) * H * 4 + B * 4 + B * _LANES * 4
                           + 2 * H * _LANES * 4),
    )(emb_self, nb_2d, inv_cnt, w_self, w_nb, b_pad)

    return out_pad[:, :2]


# speedup vs baseline: 2.4117x; 2.4117x over previous
import functools

import jax
import jax.numpy as jnp
from jax.experimental import pallas as pl
from jax.experimental.pallas import tpu as pltpu

_LANES = 128


def _head_kernel(self_ref, nb_ref, inv_ref, ws_ref, wn_ref, b_ref, out_ref,
                 *, n_nb, hidden):
    parts = [nb_ref[:, n, :] for n in range(n_nb)]
    while len(parts) > 1:
        nxt = [parts[i] + parts[i + 1] for i in range(0, len(parts) - 1, 2)]
        if len(parts) % 2:
            nxt.append(parts[-1])
        parts = nxt
    nb_sum = parts[0]

    acc = jnp.dot(self_ref[...], ws_ref[...],
                  preferred_element_type=jnp.float32)
    acc += jnp.dot(nb_sum, wn_ref[...],
                   preferred_element_type=jnp.float32) * inv_ref[...]
    out_ref[...] = acc + b_ref[...]


def kernel(emb_self, emb_nb, weight, bias, nb_counts):
    B, H = emb_self.shape
    _, N, _ = emb_nb.shape

    wt = weight.astype(jnp.float32)
    w_self = jnp.zeros((H, _LANES), jnp.float32).at[:, :2].set(wt[:, :H].T)
    w_nb = jnp.zeros((H, _LANES), jnp.float32).at[:, :2].set(wt[:, H:].T)
    b_pad = jnp.zeros((1, _LANES), jnp.float32).at[0, :2].set(
        bias.astype(jnp.float32))

    inv_cnt = (1.0 / jnp.maximum(nb_counts.astype(jnp.float32), 1.0)
               ).reshape(B, 1)

    bt = min(B, 256)
    grid = (pl.cdiv(B, bt),)

    out_pad = pl.pallas_call(
        functools.partial(_head_kernel, n_nb=N, hidden=H),
        out_shape=jax.ShapeDtypeStruct((B, _LANES), jnp.float32),
        grid_spec=pl.GridSpec(
            grid=grid,
            in_specs=[
                pl.BlockSpec((bt, H), lambda i: (i, 0)),
                pl.BlockSpec((bt, N, H), lambda i: (i, 0, 0)),
                pl.BlockSpec((bt, 1), lambda i: (i, 0)),
                pl.BlockSpec((H, _LANES), lambda i: (0, 0)),
                pl.BlockSpec((H, _LANES), lambda i: (0, 0)),
                pl.BlockSpec((1, _LANES), lambda i: (0, 0)),
            ],
            out_specs=pl.BlockSpec((bt, _LANES), lambda i: (i, 0)),
        ),
        compiler_params=pltpu.CompilerParams(
            dimension_semantics=("parallel",),
            vmem_limit_bytes=64 << 20),
        cost_estimate=pl.CostEstimate(
            flops=2 * B * H * _LANES * 2 + B * (N - 1) * H,
            transcendentals=0,
            bytes_accessed=B * (N + 1) * H * 4 + B * 4 + B * _LANES * 4
                           + 2 * H * _LANES * 4),
    )(emb_self, emb_nb, inv_cnt, w_self, w_nb, b_pad)

    return out_pad[:, :2]


# speedup vs baseline: 2.4500x; 1.0159x over previous
import functools

import jax
import jax.numpy as jnp
from jax.experimental import pallas as pl
from jax.experimental.pallas import tpu as pltpu

_LANES = 128


def _head_kernel(self_ref, nb_ref, inv_ref, ws_ref, wn_ref, b_ref, out_ref,
                 *, n_nb, hidden):
    parts = [nb_ref[:, n, :] for n in range(n_nb)]
    while len(parts) > 1:
        nxt = [parts[i] + parts[i + 1] for i in range(0, len(parts) - 1, 2)]
        if len(parts) % 2:
            nxt.append(parts[-1])
        parts = nxt
    nb_sum = parts[0]

    acc = jnp.dot(self_ref[...], ws_ref[...],
                  preferred_element_type=jnp.float32)
    acc += jnp.dot(nb_sum, wn_ref[...],
                   preferred_element_type=jnp.float32) * inv_ref[...]
    out_ref[...] = acc + b_ref[...]


def kernel(emb_self, emb_nb, weight, bias, nb_counts):
    B, H = emb_self.shape
    _, N, _ = emb_nb.shape

    wt = weight.astype(jnp.float32)
    w_self = jnp.zeros((H, _LANES), jnp.float32).at[:, :2].set(wt[:, :H].T)
    w_nb = jnp.zeros((H, _LANES), jnp.float32).at[:, :2].set(wt[:, H:].T)
    b_pad = jnp.zeros((1, _LANES), jnp.float32).at[0, :2].set(
        bias.astype(jnp.float32))

    inv_cnt = (1.0 / jnp.maximum(nb_counts.astype(jnp.float32), 1.0)
               ).reshape(B, 1)

    bt = min(B, 512)
    grid = (pl.cdiv(B, bt),)

    out_pad = pl.pallas_call(
        functools.partial(_head_kernel, n_nb=N, hidden=H),
        out_shape=jax.ShapeDtypeStruct((B, _LANES), jnp.float32),
        grid_spec=pl.GridSpec(
            grid=grid,
            in_specs=[
                pl.BlockSpec((bt, H), lambda i: (i, 0)),
                pl.BlockSpec((bt, N, H), lambda i: (i, 0, 0)),
                pl.BlockSpec((bt, 1), lambda i: (i, 0)),
                pl.BlockSpec((H, _LANES), lambda i: (0, 0)),
                pl.BlockSpec((H, _LANES), lambda i: (0, 0)),
                pl.BlockSpec((1, _LANES), lambda i: (0, 0)),
            ],
            out_specs=pl.BlockSpec((bt, _LANES), lambda i: (i, 0)),
        ),
        compiler_params=pltpu.CompilerParams(
            dimension_semantics=("parallel",),
            vmem_limit_bytes=64 << 20),
        cost_estimate=pl.CostEstimate(
            flops=2 * B * H * _LANES * 2 + B * (N - 1) * H,
            transcendentals=0,
            bytes_accessed=B * (N + 1) * H * 4 + B * 4 + B * _LANES * 4
                           + 2 * H * _LANES * 4),
    )(emb_self, emb_nb, inv_cnt, w_self, w_nb, b_pad)

    return out_pad[:, :2]
